# baseline (speedup 1.0000x reference)
import functools

import jax
import jax.numpy as jnp
from jax import lax
from jax.experimental import pallas as pl
from jax.experimental.pallas import tpu as pltpu
from jax.experimental.pallas import tpu_sc as plsc

N = 100000
E = 1600000
NUM_GRAPHS = 64
H = 48
BN = 400
GRID = N // BN
CHUNK = 128
NSUB = 16
E16 = 102400
EPAD = E16 * NSUB
NITER = E16 // CHUNK
GROUPS = NITER
DEPTH = 4
IDXD = 2 * DEPTH
GLEAD = 3
ILEAD = 4
RPS = 6400
NPAD = RPS * NSUB
FC = 256
FITER = RPS // FC


def _sc_body(do_deg, *args):
  if do_deg:
    (hA, hB, hC, srcp, dst2d, aggA, aggB, aggC, degO,
     acc, sidx, didx, rows, fbuf, *sems) = args
  else:
    (hA, hB, hC, srcp, dst2d, aggA, aggB, aggC,
     acc, sidx, didx, rows, fbuf, *sems) = args
    degO = None
  c = lax.axis_index("c")
  s = lax.axis_index("s")
  isem = sems[:IDXD]
  gsem = sems[IDXD:IDXD + DEPTH]
  ssem = sems[IDXD + DEPTH:]

  def fill_fbuf(val):
    def body(r, _):
      fbuf[r] = jnp.full((16,), val, jnp.float32)
      return 0
    lax.fori_loop(0, FC, body, 0)

  def zero_acc():
    fill_fbuf(0.0)
    def body(t, _):
      base = s * RPS + t * FC
      pltpu.sync_copy(fbuf, acc.at[pl.ds(base, FC)])
      return 0
    lax.fori_loop(0, FITER, body, 0)

  def idx_args(g, p):
    return [(srcp.at[pl.ds(s * E16 + g * CHUNK, CHUNK)], sidx.at[p], isem[p]),
            (dst2d.at[pl.ds(s * NITER + g, 1)], didx.at[p], isem[p])]

  def load_idx(g, p):
    for a in idx_args(g, p):
      pltpu.async_copy(*a)

  def drain_idx(g, p):
    for a in idx_args(g, p):
      pltpu.make_async_copy(*a).wait()

  def gat_args(h_hbm, p, ip):
    return (h_hbm.at[sidx.at[ip]], rows.at[p], gsem[p])

  def sca_args(p, ip):
    return (rows.at[p], acc.at[didx.at[ip, 0]], ssem[p])

  def edge_pass(h_hbm):
    if h_hbm is None:
      for p in range(DEPTH):
        def body(r, _, p=p):
          rows[p, r] = jnp.full((16,), 1.0, jnp.float32)
          return 0
        lax.fori_loop(0, CHUNK, body, 0)
    for g0 in range(DEPTH):
      load_idx(g0, g0)
    for g0 in range(GLEAD):
      drain_idx(g0, g0)
      if h_hbm is not None:
        pltpu.async_copy(*gat_args(h_hbm, g0, g0))

    def outer(G, _):
      for u in range(IDXD):
        g = IDXD * G + u
        r = u % DEPTH
        qg = (u + GLEAD) % DEPTH
        ig = (u + GLEAD) % IDXD
        i4 = (u + ILEAD) % IDXD

        @pl.when(jnp.logical_and(g + GLEAD < GROUPS, g + GLEAD >= DEPTH))
        def _():
          pltpu.make_async_copy(*sca_args(qg, 0)).wait()

        @pl.when(g + GLEAD < GROUPS)
        def _():
          drain_idx(g + GLEAD, ig)
          if h_hbm is not None:
            pltpu.async_copy(*gat_args(h_hbm, qg, ig))

        if h_hbm is not None:
          pltpu.make_async_copy(*gat_args(h_hbm, r, u)).wait()
        pltpu.async_copy(*sca_args(r, u), add=True)

        @pl.when(g + ILEAD < GROUPS)
        def _():
          load_idx(g + ILEAD, i4)
      return 0
    lax.fori_loop(0, GROUPS // IDXD, outer, 0)
    for p in range(DEPTH):
      pltpu.make_async_copy(*sca_args(p, 0)).wait()

  def gather_pass(h_hbm):
    edge_pass(h_hbm)

  def deg_pass():
    edge_pass(None)

  def flush(out_hbm):
    def body(t, _):
      base = s * RPS + t * FC
      pltpu.sync_copy(acc.at[pl.ds(base, FC)], fbuf)
      pltpu.sync_copy(fbuf, out_hbm.at[pl.ds(base, FC)])
      return 0
    lax.fori_loop(0, FITER, body, 0)

  zero_acc()
  plsc.subcore_barrier()
  pl.when(c == 0)(lambda: gather_pass(hA))
  pl.when(c == 1)(lambda: gather_pass(hB))
  plsc.subcore_barrier()
  pl.when(c == 0)(lambda: flush(aggA))
  pl.when(c == 1)(lambda: flush(aggB))
  plsc.subcore_barrier()
  zero_acc()
  plsc.subcore_barrier()
  pl.when(c == 0)(lambda: gather_pass(hC))
  if do_deg:
    pl.when(c == 1)(deg_pass)
  plsc.subcore_barrier()
  pl.when(c == 0)(lambda: flush(aggC))
  if do_deg:
    pl.when(c == 1)(lambda: flush(degO))


def _make_sc(do_deg):
  outs = [jax.ShapeDtypeStruct((NPAD, 16), jnp.float32)] * (4 if do_deg else 3)
  return pl.kernel(
      functools.partial(_sc_body, do_deg),
      out_type=outs,
      mesh=plsc.VectorSubcoreMesh(core_axis_name="c", subcore_axis_name="s"),
      compiler_params=pltpu.CompilerParams(use_tc_tiling_on_sc=False),
      scratch_types=[
          pltpu.VMEM_SHARED((NPAD, 16), jnp.float32),
          pltpu.VMEM((IDXD, CHUNK), jnp.int32),
          pltpu.VMEM((IDXD, 1, CHUNK), jnp.int32),
          pltpu.VMEM((DEPTH, CHUNK, 16), jnp.float32),
          pltpu.VMEM((FC, 16), jnp.float32),
      ] + [pltpu.SemaphoreType.DMA] * (IDXD + 2 * DEPTH),
  )


def _k1_body(x_ref, xd_ref, xst_ref, emb_ref, h0_ref, hA_ref, hB_ref, hC_ref):
  xst = xst_ref[...]
  oh = (lax.broadcasted_iota(jnp.int32, (BN, 256), 1) == xst).astype(jnp.float32)
  emb = jnp.dot(oh, emb_ref[...], preferred_element_type=jnp.float32)
  h = jnp.concatenate(
      [x_ref[...], xd_ref[...], emb, jnp.zeros((BN, 2), jnp.float32)], axis=1)
  h0_ref[...] = h
  hA_ref[...] = h[:, 0:16]
  hB_ref[...] = h[:, 16:32]
  hC_ref[...] = h[:, 32:48]


def _combine(aA, aB, aC, dg, h, WlT, bl, WrT, g, b):
  agg = jnp.concatenate([aA[...], aB[...], aC[...]], axis=1)
  invd = 1.0 / jnp.maximum(dg[...][:, 0:1], 1.0)
  z = (jnp.dot(agg * invd, WlT[...], preferred_element_type=jnp.float32)
       + bl[...]
       + jnp.dot(h[...], WrT[...], preferred_element_type=jnp.float32))
  mu = jnp.mean(z, axis=1, keepdims=True)
  var = jnp.mean((z - mu) ** 2, axis=1, keepdims=True)
  zn = (z - mu) * lax.rsqrt(var + 1e-5) * g[...] + b[...]
  return jnp.maximum(zn, 0.0)


def _k2_body(aA, aB, aC, dg, h0, WlT, bl, WrT, g, b,
             h1_ref, oA_ref, oB_ref, oC_ref):
  h1 = _combine(aA, aB, aC, dg, h0, WlT, bl, WrT, g, b)
  h1_ref[...] = h1
  oA_ref[...] = h1[:, 0:16]
  oB_ref[...] = h1[:, 16:32]
  oC_ref[...] = h1[:, 32:48]


def _k3_body(aA, aB, aC, dg, h1, WlT, bl, WrT, g, b, bt_ref,
             W1T, b1, W2T, b2, W3T, b3, out_ref, ssum, smax, scnt):
  i = pl.program_id(0)

  @pl.when(i == 0)
  def _init():
    ssum[...] = jnp.zeros((NUM_GRAPHS, H), jnp.float32)
    smax[...] = jnp.full((NUM_GRAPHS, H), -jnp.inf, jnp.float32)
    scnt[...] = jnp.zeros((NUM_GRAPHS, 1), jnp.float32)

  h2 = _combine(aA, aB, aC, dg, h1, WlT, bl, WrT, g, b)
  bt = bt_ref[...]
  m = bt == lax.broadcasted_iota(jnp.int32, (BN, NUM_GRAPHS), 1)
  mf = m.astype(jnp.float32)
  ssum[...] += lax.dot_general(mf, h2, (((0,), (0,)), ((), ())),
                               preferred_element_type=jnp.float32)
  scnt[...] += jnp.sum(mf, axis=0).reshape(NUM_GRAPHS, 1)
  for gi in range(NUM_GRAPHS):
    v = jnp.max(jnp.where(m[:, gi:gi + 1], h2, -jnp.inf), axis=0)
    smax[gi:gi + 1, :] = jnp.maximum(smax[gi:gi + 1, :], v.reshape(1, H))

  @pl.when(i == GRID - 1)
  def _fin():
    mean = ssum[...] / jnp.maximum(scnt[...], 1.0)
    zz = jnp.concatenate([mean, smax[...]], axis=1)
    zz = jnp.maximum(jnp.dot(zz, W1T[...], preferred_element_type=jnp.float32) + b1[...], 0.0)
    zz = jnp.maximum(jnp.dot(zz, W2T[...], preferred_element_type=jnp.float32) + b2[...], 0.0)
    zz = jnp.maximum(jnp.dot(zz, W3T[...], preferred_element_type=jnp.float32) + b3[...], 0.0)
    out_ref[...] = zz


def _row_spec(w):
  return pl.BlockSpec((BN, w), lambda i: (i, 0))


def _full_spec(shape):
  return pl.BlockSpec(shape, lambda i: tuple(0 for _ in shape))


def kernel(x, edge_index, edge_attr, xdims, xsttype, batch, st_emb,
           Wl1, bl1, Wr1, g1, beta1, Wl2, bl2, Wr2, g2, beta2,
           Wfc1, bfc1, Wfc2, bfc2, Wfc3, bfc3):
  f32 = jnp.float32
  srcp = jnp.concatenate([edge_index[0], jnp.zeros((EPAD - E,), jnp.int32)])
  dstp = jnp.concatenate([edge_index[1], jnp.full((EPAD - E,), N, jnp.int32)])
  dst2d = dstp.reshape(EPAD // CHUNK, CHUNK)
  Wl1T = jnp.pad(Wl1, ((0, 0), (0, 2))).T
  Wr1T = jnp.pad(Wr1, ((0, 0), (0, 2))).T
  Wl2T = Wl2.T
  Wr2T = Wr2.T
  row = lambda v: v.reshape(1, -1)

  k1 = pl.pallas_call(
      _k1_body,
      grid=(GRID,),
      in_specs=[_row_spec(32), _row_spec(2), _row_spec(1), _full_spec((256, 12))],
      out_specs=[_row_spec(48), _row_spec(16), _row_spec(16), _row_spec(16)],
      out_shape=[jax.ShapeDtypeStruct((N, 48), f32)] +
                [jax.ShapeDtypeStruct((N, 16), f32)] * 3,
  )
  h0, hA, hB, hC = k1(x, xdims, xsttype.reshape(N, 1), st_emb)

  sc1 = _make_sc(True)
  aggA, aggB, aggC, deg = sc1(hA, hB, hC, srcp, dst2d)

  wspec = [_full_spec((48, 48)), _full_spec((1, 48)), _full_spec((48, 48)),
           _full_spec((1, 48)), _full_spec((1, 48))]
  k2 = pl.pallas_call(
      _k2_body,
      grid=(GRID,),
      in_specs=[_row_spec(16)] * 3 + [_row_spec(16), _row_spec(48)] + wspec,
      out_specs=[_row_spec(48), _row_spec(16), _row_spec(16), _row_spec(16)],
      out_shape=[jax.ShapeDtypeStruct((N, 48), f32)] +
                [jax.ShapeDtypeStruct((N, 16), f32)] * 3,
  )
  h1, h1A, h1B, h1C = k2(aggA, aggB, aggC, deg, h0,
                         Wl1T, row(bl1), Wr1T, row(g1), row(beta1))

  sc2 = _make_sc(False)
  agg2A, agg2B, agg2C = sc2(h1A, h1B, h1C, srcp, dst2d)

  k3 = pl.pallas_call(
      _k3_body,
      grid=(GRID,),
      in_specs=([_row_spec(16)] * 3 + [_row_spec(16), _row_spec(48)] + wspec +
                [_row_spec(1),
                 _full_spec((96, 50)), _full_spec((1, 50)),
                 _full_spec((50, 50)), _full_spec((1, 50)),
                 _full_spec((50, 10)), _full_spec((1, 10))]),
      out_specs=pl.BlockSpec((NUM_GRAPHS, 10), lambda i: (0, 0)),
      out_shape=jax.ShapeDtypeStruct((NUM_GRAPHS, 10), f32),
      scratch_shapes=[pltpu.VMEM((NUM_GRAPHS, H), f32),
                      pltpu.VMEM((NUM_GRAPHS, H), f32),
                      pltpu.VMEM((NUM_GRAPHS, 1), f32)],
  )
  out = k3(agg2A, agg2B, agg2C, deg, h1,
           Wl2T, row(bl2), Wr2T, row(g2), row(beta2), batch.reshape(N, 1),
           Wfc1.T, row(bfc1), Wfc2.T, row(bfc2), Wfc3.T, row(bfc3))
  return out


# speedup vs baseline: 1.1286x; 1.1286x over previous
import functools

import jax
import jax.numpy as jnp
from jax import lax
from jax.experimental import pallas as pl
from jax.experimental.pallas import tpu as pltpu
from jax.experimental.pallas import tpu_sc as plsc

N = 100000
E = 1600000
NUM_GRAPHS = 64
H = 48
BN = 400
GRID = N // BN
CHUNK = 128
NSUB = 16
E16 = 102400
EPAD = E16 * NSUB
NITER = E16 // CHUNK
GROUPS = NITER
DEPTH = 4
IDXD = 2 * DEPTH
GLEAD = 2
ILEAD = 4
RPS = 6400
NPAD = RPS * NSUB
FC = 256
FITER = RPS // FC


def _sc_body(schedule, n_out, *args):
  (hA, hB, hC, srcp, dst2d) = args[:5]
  outs = args[5:5 + n_out]
  (acc, sidx, didx, rows, fbuf, *sems) = args[5 + n_out:]
  hs = [hA, hB, hC]
  c = lax.axis_index("c")
  s = lax.axis_index("s")
  isem = sems[:IDXD]
  gsem = sems[IDXD:IDXD + DEPTH]
  ssem = sems[IDXD + DEPTH:]

  def fill_fbuf(val):
    def body(r, _):
      fbuf[r] = jnp.full((16,), val, jnp.float32)
      return 0
    lax.fori_loop(0, FC, body, 0)

  def zero_acc():
    fill_fbuf(0.0)
    def body(t, _):
      base = s * RPS + t * FC
      pltpu.sync_copy(fbuf, acc.at[pl.ds(base, FC)])
      return 0
    lax.fori_loop(0, FITER, body, 0)

  def idx_args(g, p):
    return [(srcp.at[pl.ds(s * E16 + g * CHUNK, CHUNK)], sidx.at[p], isem[p]),
            (dst2d.at[pl.ds(s * NITER + g, 1)], didx.at[p], isem[p])]

  def load_idx(g, p):
    for a in idx_args(g, p):
      pltpu.async_copy(*a)

  def drain_idx(g, p):
    for a in idx_args(g, p):
      pltpu.make_async_copy(*a).wait()

  def gat_args(h_hbm, p, ip):
    return (h_hbm.at[sidx.at[ip]], rows.at[p], gsem[p])

  def sca_args(p, ip):
    return (rows.at[p], acc.at[didx.at[ip, 0]], ssem[p])

  def edge_pass(h_hbm, glo, ng):
    if h_hbm is None:
      for p in range(DEPTH):
        def body(r, _, p=p):
          rows[p, r] = jnp.full((16,), 1.0, jnp.float32)
          return 0
        lax.fori_loop(0, CHUNK, body, 0)
    for g0 in range(DEPTH):
      load_idx(glo + g0, g0)
    for g0 in range(GLEAD):
      drain_idx(glo + g0, g0)
      if h_hbm is not None:
        pltpu.async_copy(*gat_args(h_hbm, g0, g0))

    def outer(G, _):
      for u in range(IDXD):
        g = IDXD * G + u
        r = u % DEPTH
        qg = (u + GLEAD) % DEPTH
        ig = (u + GLEAD) % IDXD
        i4 = (u + ILEAD) % IDXD

        @pl.when(jnp.logical_and(g + GLEAD < ng, g + GLEAD >= DEPTH))
        def _():
          pltpu.make_async_copy(*sca_args(qg, 0)).wait()

        @pl.when(g + GLEAD < ng)
        def _():
          drain_idx(glo + g + GLEAD, ig)
          if h_hbm is not None:
            pltpu.async_copy(*gat_args(h_hbm, qg, ig))

        if h_hbm is not None:
          pltpu.make_async_copy(*gat_args(h_hbm, r, u)).wait()
        pltpu.async_copy(*sca_args(r, u), add=True)

        @pl.when(g + ILEAD < ng)
        def _():
          load_idx(glo + g + ILEAD, i4)
      return 0
    lax.fori_loop(0, ng // IDXD, outer, 0)
    for p in range(DEPTH):
      pltpu.make_async_copy(*sca_args(p, 0)).wait()

  def flush(out_hbm):
    def body(t, _):
      base = s * RPS + t * FC
      pltpu.sync_copy(acc.at[pl.ds(base, FC)], fbuf)
      pltpu.sync_copy(fbuf, out_hbm.at[pl.ds(base, FC)])
      return 0
    lax.fori_loop(0, FITER, body, 0)

  for core_id in range(2):
    def run_core(core_id=core_id):
      for (h_idx, out_idx, glo, ng) in schedule[core_id]:
        zero_acc()
        plsc.subcore_barrier()
        edge_pass(None if h_idx is None else hs[h_idx], glo, ng)
        plsc.subcore_barrier()
        flush(outs[out_idx])
        plsc.subcore_barrier()
    pl.when(c == core_id)(run_core)


def _make_sc(schedule, n_out):
  outs = [jax.ShapeDtypeStruct((NPAD, 16), jnp.float32)] * n_out
  return pl.kernel(
      functools.partial(_sc_body, schedule, n_out),
      out_type=outs,
      mesh=plsc.VectorSubcoreMesh(core_axis_name="c", subcore_axis_name="s"),
      compiler_params=pltpu.CompilerParams(use_tc_tiling_on_sc=False),
      scratch_types=[
          pltpu.VMEM_SHARED((NPAD, 16), jnp.float32),
          pltpu.VMEM((IDXD, CHUNK), jnp.int32),
          pltpu.VMEM((IDXD, 1, CHUNK), jnp.int32),
          pltpu.VMEM((DEPTH, CHUNK, 16), jnp.float32),
          pltpu.VMEM((FC, 16), jnp.float32),
      ] + [pltpu.SemaphoreType.DMA] * (IDXD + 2 * DEPTH),
  )


def _k1_body(x_ref, xd_ref, xst_ref, emb_ref, h0_ref, hA_ref, hB_ref, hC_ref):
  xst = xst_ref[...]
  oh = (lax.broadcasted_iota(jnp.int32, (BN, 256), 1) == xst).astype(jnp.float32)
  emb = jnp.dot(oh, emb_ref[...], preferred_element_type=jnp.float32)
  h = jnp.concatenate(
      [x_ref[...], xd_ref[...], emb, jnp.zeros((BN, 2), jnp.float32)], axis=1)
  h0_ref[...] = h
  hA_ref[...] = h[:, 0:16]
  hB_ref[...] = h[:, 16:32]
  hC_ref[...] = h[:, 32:48]


def _combine(aA, aB, aC0, aC1, dg, h, WlT, bl, WrT, g, b):
  agg = jnp.concatenate([aA[...], aB[...], aC0[...] + aC1[...]], axis=1)
  invd = 1.0 / jnp.maximum(dg[...][:, 0:1], 1.0)
  z = (jnp.dot(agg * invd, WlT[...], preferred_element_type=jnp.float32)
       + bl[...]
       + jnp.dot(h[...], WrT[...], preferred_element_type=jnp.float32))
  mu = jnp.mean(z, axis=1, keepdims=True)
  var = jnp.mean((z - mu) ** 2, axis=1, keepdims=True)
  zn = (z - mu) * lax.rsqrt(var + 1e-5) * g[...] + b[...]
  return jnp.maximum(zn, 0.0)


def _k2_body(aA, aB, aC0, aC1, dg, h0, WlT, bl, WrT, g, b,
             h1_ref, oA_ref, oB_ref, oC_ref):
  h1 = _combine(aA, aB, aC0, aC1, dg, h0, WlT, bl, WrT, g, b)
  h1_ref[...] = h1
  oA_ref[...] = h1[:, 0:16]
  oB_ref[...] = h1[:, 16:32]
  oC_ref[...] = h1[:, 32:48]


def _k3_body(aA, aB, aC0, aC1, dg, h1, WlT, bl, WrT, g, b, bt_ref,
             W1T, b1, W2T, b2, W3T, b3, out_ref, ssum, smax, scnt):
  i = pl.program_id(0)

  @pl.when(i == 0)
  def _init():
    ssum[...] = jnp.zeros((NUM_GRAPHS, H), jnp.float32)
    smax[...] = jnp.full((NUM_GRAPHS, H), -jnp.inf, jnp.float32)
    scnt[...] = jnp.zeros((NUM_GRAPHS, 1), jnp.float32)

  h2 = _combine(aA, aB, aC0, aC1, dg, h1, WlT, bl, WrT, g, b)
  bt = bt_ref[...]
  m = bt == lax.broadcasted_iota(jnp.int32, (BN, NUM_GRAPHS), 1)
  mf = m.astype(jnp.float32)
  ssum[...] += lax.dot_general(mf, h2, (((0,), (0,)), ((), ())),
                               preferred_element_type=jnp.float32)
  scnt[...] += jnp.sum(mf, axis=0).reshape(NUM_GRAPHS, 1)
  for gi in range(NUM_GRAPHS):
    v = jnp.max(jnp.where(m[:, gi:gi + 1], h2, -jnp.inf), axis=0)
    smax[gi:gi + 1, :] = jnp.maximum(smax[gi:gi + 1, :], v.reshape(1, H))

  @pl.when(i == GRID - 1)
  def _fin():
    mean = ssum[...] / jnp.maximum(scnt[...], 1.0)
    zz = jnp.concatenate([mean, smax[...]], axis=1)
    zz = jnp.maximum(jnp.dot(zz, W1T[...], preferred_element_type=jnp.float32) + b1[...], 0.0)
    zz = jnp.maximum(jnp.dot(zz, W2T[...], preferred_element_type=jnp.float32) + b2[...], 0.0)
    zz = jnp.maximum(jnp.dot(zz, W3T[...], preferred_element_type=jnp.float32) + b3[...], 0.0)
    out_ref[...] = zz


def _row_spec(w):
  return pl.BlockSpec((BN, w), lambda i: (i, 0))


def _full_spec(shape):
  return pl.BlockSpec(shape, lambda i: tuple(0 for _ in shape))


def kernel(x, edge_index, edge_attr, xdims, xsttype, batch, st_emb,
           Wl1, bl1, Wr1, g1, beta1, Wl2, bl2, Wr2, g2, beta2,
           Wfc1, bfc1, Wfc2, bfc2, Wfc3, bfc3):
  f32 = jnp.float32
  srcp = jnp.concatenate([edge_index[0], jnp.zeros((EPAD - E,), jnp.int32)])
  dstp = jnp.concatenate([edge_index[1], jnp.full((EPAD - E,), N, jnp.int32)])
  dst2d = dstp.reshape(EPAD // CHUNK, CHUNK)
  Wl1T = jnp.pad(Wl1, ((0, 0), (0, 2))).T
  Wr1T = jnp.pad(Wr1, ((0, 0), (0, 2))).T
  Wl2T = Wl2.T
  Wr2T = Wr2.T
  row = lambda v: v.reshape(1, -1)

  k1 = pl.pallas_call(
      _k1_body,
      grid=(GRID,),
      in_specs=[_row_spec(32), _row_spec(2), _row_spec(1), _full_spec((256, 12))],
      out_specs=[_row_spec(48), _row_spec(16), _row_spec(16), _row_spec(16)],
      out_shape=[jax.ShapeDtypeStruct((N, 48), f32)] +
                [jax.ShapeDtypeStruct((N, 16), f32)] * 3,
  )
  h0, hA, hB, hC = k1(x, xdims, xsttype.reshape(N, 1), st_emb)

  sched1 = (((0, 0, 0, 800), (2, 2, 0, 600)),
            ((1, 1, 0, 800), (None, 4, 0, 800), (2, 3, 600, 200)))
  sc1 = _make_sc(sched1, 5)
  aggA, aggB, aggC0, aggC1, deg = sc1(hA, hB, hC, srcp, dst2d)

  wspec = [_full_spec((48, 48)), _full_spec((1, 48)), _full_spec((48, 48)),
           _full_spec((1, 48)), _full_spec((1, 48))]
  k2 = pl.pallas_call(
      _k2_body,
      grid=(GRID,),
      in_specs=[_row_spec(16)] * 4 + [_row_spec(16), _row_spec(48)] + wspec,
      out_specs=[_row_spec(48), _row_spec(16), _row_spec(16), _row_spec(16)],
      out_shape=[jax.ShapeDtypeStruct((N, 48), f32)] +
                [jax.ShapeDtypeStruct((N, 16), f32)] * 3,
  )
  h1, h1A, h1B, h1C = k2(aggA, aggB, aggC0, aggC1, deg, h0,
                         Wl1T, row(bl1), Wr1T, row(g1), row(beta1))

  sched2 = (((0, 0, 0, 800), (2, 2, 0, 400)),
            ((1, 1, 0, 800), (2, 3, 400, 400)))
  sc2 = _make_sc(sched2, 4)
  agg2A, agg2B, agg2C0, agg2C1 = sc2(h1A, h1B, h1C, srcp, dst2d)

  k3 = pl.pallas_call(
      _k3_body,
      grid=(GRID,),
      in_specs=([_row_spec(16)] * 4 + [_row_spec(16), _row_spec(48)] + wspec +
                [_row_spec(1),
                 _full_spec((96, 50)), _full_spec((1, 50)),
                 _full_spec((50, 50)), _full_spec((1, 50)),
                 _full_spec((50, 10)), _full_spec((1, 10))]),
      out_specs=pl.BlockSpec((NUM_GRAPHS, 10), lambda i: (0, 0)),
      out_shape=jax.ShapeDtypeStruct((NUM_GRAPHS, 10), f32),
      scratch_shapes=[pltpu.VMEM((NUM_GRAPHS, H), f32),
                      pltpu.VMEM((NUM_GRAPHS, H), f32),
                      pltpu.VMEM((NUM_GRAPHS, 1), f32)],
  )
  out = k3(agg2A, agg2B, agg2C0, agg2C1, deg, h1,
           Wl2T, row(bl2), Wr2T, row(g2), row(beta2), batch.reshape(N, 1),
           Wfc1.T, row(bfc1), Wfc2.T, row(bfc2), Wfc3.T, row(bfc3))
  return out


# speedup vs baseline: 1.1735x; 1.0397x over previous
import functools

import jax
import jax.numpy as jnp
from jax import lax
from jax.experimental import pallas as pl
from jax.experimental.pallas import tpu as pltpu
from jax.experimental.pallas import tpu_sc as plsc

N = 100000
E = 1600000
NUM_GRAPHS = 64
H = 48
BN = 400
GRID = N // BN
CHUNK = 128
NSUB = 16
E16 = 102400
EPAD = E16 * NSUB
NITER = E16 // CHUNK
CROWS = 2
GEDGE = CROWS * CHUNK
GROUPS = NITER // CROWS
DEPTH = 4
IDXD = 2 * DEPTH
GLEAD = 2
ILEAD = 4
RPS = 6400
NPAD = RPS * NSUB
FC = 128
FITER = RPS // FC


def _sc_body(schedule, n_out, *args):
  (hA, hB, hC, src2d, dst2d) = args[:5]
  outs = args[5:5 + n_out]
  (acc, sidx, didx, rows, fbuf, *sems) = args[5 + n_out:]
  hs = [hA, hB, hC]
  c = lax.axis_index("c")
  s = lax.axis_index("s")
  isem = sems[:IDXD]
  gsem = sems[IDXD:IDXD + DEPTH]
  ssem = sems[IDXD + DEPTH:]

  def fill_fbuf(val):
    def body(r, _):
      fbuf[r] = jnp.full((16,), val, jnp.float32)
      return 0
    lax.fori_loop(0, FC, body, 0)

  def zero_acc():
    fill_fbuf(0.0)
    def body(t, _):
      base = s * RPS + t * FC
      pltpu.sync_copy(fbuf, acc.at[pl.ds(base, FC)])
      return 0
    lax.fori_loop(0, FITER, body, 0)

  def idx_args(g, p):
    return [(src2d.at[pl.ds(s * E16 + g * GEDGE, GEDGE)], sidx.at[p], isem[p]),
            (dst2d.at[pl.ds(s * E16 + g * GEDGE, GEDGE)], didx.at[p], isem[p])]

  def load_idx(g, p):
    for a in idx_args(g, p):
      pltpu.async_copy(*a)

  def drain_idx(g, p):
    for a in idx_args(g, p):
      pltpu.make_async_copy(*a).wait()

  def gat_args(h_hbm, p, ip):
    return (h_hbm.at[sidx.at[ip]], rows.at[p], gsem[p])

  def sca_args(p, ip):
    return (rows.at[p], acc.at[didx.at[ip]], ssem[p])

  def edge_pass(h_hbm, glo, ng):
    if h_hbm is None:
      for p in range(DEPTH):
        def body(r, _, p=p):
          rows[p, r] = jnp.full((16,), 1.0, jnp.float32)
          return 0
        lax.fori_loop(0, GEDGE, body, 0)
    for g0 in range(DEPTH):
      load_idx(glo + g0, g0)
    for g0 in range(GLEAD):
      drain_idx(glo + g0, g0)
      if h_hbm is not None:
        pltpu.async_copy(*gat_args(h_hbm, g0, g0))

    def outer(G, _):
      for u in range(IDXD):
        g = IDXD * G + u
        r = u % DEPTH
        qg = (u + GLEAD) % DEPTH
        ig = (u + GLEAD) % IDXD
        i4 = (u + ILEAD) % IDXD

        @pl.when(jnp.logical_and(g + GLEAD < ng, g + GLEAD >= DEPTH))
        def _():
          pltpu.make_async_copy(*sca_args(qg, 0)).wait()

        @pl.when(g + GLEAD < ng)
        def _():
          drain_idx(glo + g + GLEAD, ig)
          if h_hbm is not None:
            pltpu.async_copy(*gat_args(h_hbm, qg, ig))

        if h_hbm is not None:
          pltpu.make_async_copy(*gat_args(h_hbm, r, u)).wait()
        pltpu.async_copy(*sca_args(r, u), add=True)

        @pl.when(g + ILEAD < ng)
        def _():
          load_idx(glo + g + ILEAD, i4)
      return 0
    lax.fori_loop(0, ng // IDXD, outer, 0)
    for p in range(DEPTH):
      pltpu.make_async_copy(*sca_args(p, 0)).wait()

  def flush(out_hbm):
    def body(t, _):
      base = s * RPS + t * FC
      pltpu.sync_copy(acc.at[pl.ds(base, FC)], fbuf)
      pltpu.sync_copy(fbuf, out_hbm.at[pl.ds(base, FC)])
      return 0
    lax.fori_loop(0, FITER, body, 0)

  for core_id in range(2):
    def run_core(core_id=core_id):
      for (h_idx, out_idx, glo, ng) in schedule[core_id]:
        zero_acc()
        plsc.subcore_barrier()
        edge_pass(None if h_idx is None else hs[h_idx], glo, ng)
        plsc.subcore_barrier()
        flush(outs[out_idx])
        plsc.subcore_barrier()
    pl.when(c == core_id)(run_core)


def _make_sc(schedule, n_out):
  outs = [jax.ShapeDtypeStruct((NPAD, 16), jnp.float32)] * n_out
  return pl.kernel(
      functools.partial(_sc_body, schedule, n_out),
      out_type=outs,
      mesh=plsc.VectorSubcoreMesh(core_axis_name="c", subcore_axis_name="s"),
      compiler_params=pltpu.CompilerParams(use_tc_tiling_on_sc=False),
      scratch_types=[
          pltpu.VMEM_SHARED((NPAD, 16), jnp.float32),
          pltpu.VMEM((IDXD, GEDGE), jnp.int32),
          pltpu.VMEM((IDXD, GEDGE), jnp.int32),
          pltpu.VMEM((DEPTH, GEDGE, 16), jnp.float32),
          pltpu.VMEM((FC, 16), jnp.float32),
      ] + [pltpu.SemaphoreType.DMA] * (IDXD + 2 * DEPTH),
  )


def _k1_body(x_ref, xd_ref, xst_ref, emb_ref, h0_ref, hA_ref, hB_ref, hC_ref):
  xst = xst_ref[...]
  oh = (lax.broadcasted_iota(jnp.int32, (BN, 256), 1) == xst).astype(jnp.float32)
  emb = jnp.dot(oh, emb_ref[...], preferred_element_type=jnp.float32)
  h = jnp.concatenate(
      [x_ref[...], xd_ref[...], emb, jnp.zeros((BN, 2), jnp.float32)], axis=1)
  h0_ref[...] = h
  hA_ref[...] = h[:, 0:16]
  hB_ref[...] = h[:, 16:32]
  hC_ref[...] = h[:, 32:48]


def _combine(aA, aB, aC0, aC1, dg, h, WlT, bl, WrT, g, b):
  agg = jnp.concatenate([aA[...], aB[...], aC0[...] + aC1[...]], axis=1)
  invd = 1.0 / jnp.maximum(dg[...][:, 0:1], 1.0)
  z = (jnp.dot(agg * invd, WlT[...], preferred_element_type=jnp.float32)
       + bl[...]
       + jnp.dot(h[...], WrT[...], preferred_element_type=jnp.float32))
  mu = jnp.mean(z, axis=1, keepdims=True)
  var = jnp.mean((z - mu) ** 2, axis=1, keepdims=True)
  zn = (z - mu) * lax.rsqrt(var + 1e-5) * g[...] + b[...]
  return jnp.maximum(zn, 0.0)


def _k2_body(aA, aB, aC0, aC1, dg, h0, WlT, bl, WrT, g, b,
             h1_ref, oA_ref, oB_ref, oC_ref):
  h1 = _combine(aA, aB, aC0, aC1, dg, h0, WlT, bl, WrT, g, b)
  h1_ref[...] = h1
  oA_ref[...] = h1[:, 0:16]
  oB_ref[...] = h1[:, 16:32]
  oC_ref[...] = h1[:, 32:48]


def _k3_body(aA, aB, aC0, aC1, dg, h1, WlT, bl, WrT, g, b, bt_ref,
             W1T, b1, W2T, b2, W3T, b3, out_ref, ssum, smax, scnt):
  i = pl.program_id(0)

  @pl.when(i == 0)
  def _init():
    ssum[...] = jnp.zeros((NUM_GRAPHS, H), jnp.float32)
    smax[...] = jnp.full((NUM_GRAPHS, H), -jnp.inf, jnp.float32)
    scnt[...] = jnp.zeros((NUM_GRAPHS, 1), jnp.float32)

  h2 = _combine(aA, aB, aC0, aC1, dg, h1, WlT, bl, WrT, g, b)
  bt = bt_ref[...]
  m = bt == lax.broadcasted_iota(jnp.int32, (BN, NUM_GRAPHS), 1)
  mf = m.astype(jnp.float32)
  ssum[...] += lax.dot_general(mf, h2, (((0,), (0,)), ((), ())),
                               preferred_element_type=jnp.float32)
  scnt[...] += jnp.sum(mf, axis=0).reshape(NUM_GRAPHS, 1)
  for gi in range(NUM_GRAPHS):
    v = jnp.max(jnp.where(m[:, gi:gi + 1], h2, -jnp.inf), axis=0)
    smax[gi:gi + 1, :] = jnp.maximum(smax[gi:gi + 1, :], v.reshape(1, H))

  @pl.when(i == GRID - 1)
  def _fin():
    mean = ssum[...] / jnp.maximum(scnt[...], 1.0)
    zz = jnp.concatenate([mean, smax[...]], axis=1)
    zz = jnp.maximum(jnp.dot(zz, W1T[...], preferred_element_type=jnp.float32) + b1[...], 0.0)
    zz = jnp.maximum(jnp.dot(zz, W2T[...], preferred_element_type=jnp.float32) + b2[...], 0.0)
    zz = jnp.maximum(jnp.dot(zz, W3T[...], preferred_element_type=jnp.float32) + b3[...], 0.0)
    out_ref[...] = zz


def _row_spec(w):
  return pl.BlockSpec((BN, w), lambda i: (i, 0))


def _full_spec(shape):
  return pl.BlockSpec(shape, lambda i: tuple(0 for _ in shape))


def kernel(x, edge_index, edge_attr, xdims, xsttype, batch, st_emb,
           Wl1, bl1, Wr1, g1, beta1, Wl2, bl2, Wr2, g2, beta2,
           Wfc1, bfc1, Wfc2, bfc2, Wfc3, bfc3):
  f32 = jnp.float32
  srcp = jnp.concatenate([edge_index[0], jnp.zeros((EPAD - E,), jnp.int32)])
  dstp = jnp.concatenate([edge_index[1], jnp.full((EPAD - E,), N, jnp.int32)])
  src2d = srcp
  dst2d = dstp
  Wl1T = jnp.pad(Wl1, ((0, 0), (0, 2))).T
  Wr1T = jnp.pad(Wr1, ((0, 0), (0, 2))).T
  Wl2T = Wl2.T
  Wr2T = Wr2.T
  row = lambda v: v.reshape(1, -1)

  k1 = pl.pallas_call(
      _k1_body,
      grid=(GRID,),
      in_specs=[_row_spec(32), _row_spec(2), _row_spec(1), _full_spec((256, 12))],
      out_specs=[_row_spec(48), _row_spec(16), _row_spec(16), _row_spec(16)],
      out_shape=[jax.ShapeDtypeStruct((N, 48), f32)] +
                [jax.ShapeDtypeStruct((N, 16), f32)] * 3,
  )
  h0, hA, hB, hC = k1(x, xdims, xsttype.reshape(N, 1), st_emb)

  sched1 = (((0, 0, 0, GROUPS), (2, 2, 0, 304)),
            ((1, 1, 0, GROUPS), (None, 4, 0, GROUPS), (2, 3, 304, 96)))
  sc1 = _make_sc(sched1, 5)
  aggA, aggB, aggC0, aggC1, deg = sc1(hA, hB, hC, src2d, dst2d)

  wspec = [_full_spec((48, 48)), _full_spec((1, 48)), _full_spec((48, 48)),
           _full_spec((1, 48)), _full_spec((1, 48))]
  k2 = pl.pallas_call(
      _k2_body,
      grid=(GRID,),
      in_specs=[_row_spec(16)] * 4 + [_row_spec(16), _row_spec(48)] + wspec,
      out_specs=[_row_spec(48), _row_spec(16), _row_spec(16), _row_spec(16)],
      out_shape=[jax.ShapeDtypeStruct((N, 48), f32)] +
                [jax.ShapeDtypeStruct((N, 16), f32)] * 3,
  )
  h1, h1A, h1B, h1C = k2(aggA, aggB, aggC0, aggC1, deg, h0,
                         Wl1T, row(bl1), Wr1T, row(g1), row(beta1))

  sched2 = (((0, 0, 0, GROUPS), (2, 2, 0, 200)),
            ((1, 1, 0, GROUPS), (2, 3, 200, 200)))
  sc2 = _make_sc(sched2, 4)
  agg2A, agg2B, agg2C0, agg2C1 = sc2(h1A, h1B, h1C, src2d, dst2d)

  k3 = pl.pallas_call(
      _k3_body,
      grid=(GRID,),
      in_specs=([_row_spec(16)] * 4 + [_row_spec(16), _row_spec(48)] + wspec +
                [_row_spec(1),
                 _full_spec((96, 50)), _full_spec((1, 50)),
                 _full_spec((50, 50)), _full_spec((1, 50)),
                 _full_spec((50, 10)), _full_spec((1, 10))]),
      out_specs=pl.BlockSpec((NUM_GRAPHS, 10), lambda i: (0, 0)),
      out_shape=jax.ShapeDtypeStruct((NUM_GRAPHS, 10), f32),
      scratch_shapes=[pltpu.VMEM((NUM_GRAPHS, H), f32),
                      pltpu.VMEM((NUM_GRAPHS, H), f32),
                      pltpu.VMEM((NUM_GRAPHS, 1), f32)],
  )
  out = k3(agg2A, agg2B, agg2C0, agg2C1, deg, h1,
           Wl2T, row(bl2), Wr2T, row(g2), row(beta2), batch.reshape(N, 1),
           Wfc1.T, row(bfc1), Wfc2.T, row(bfc2), Wfc3.T, row(bfc3))
  return out


# speedup vs baseline: 1.4812x; 1.2622x over previous
import functools

import jax
import jax.numpy as jnp
from jax import lax
from jax.experimental import pallas as pl
from jax.experimental.pallas import tpu as pltpu
from jax.experimental.pallas import tpu_sc as plsc

N = 100000
E = 1600000
NUM_GRAPHS = 64
H = 48
BN = 400
GRID = N // BN
CHUNK = 128
NSUB = 16
E16 = 100352
EPAD = E16 * NSUB
NITER = E16 // CHUNK
CROWS = 2
GEDGE = CROWS * CHUNK
GROUPS = NITER // CROWS
DEPTH = 4
IDXD = 2 * DEPTH
GLEAD = 2
ILEAD = 4
RPS = 6400
NPAD = RPS * NSUB
FC = 128
FITER = RPS // FC


def _sc_body(schedule, n_out, *args):
  (hA, hB, hC, src2d, dst2d) = args[:5]
  outs = args[5:5 + n_out]
  (acc, sidx, didx, rows, fbuf, *sems) = args[5 + n_out:]
  hs = [hA, hB, hC]
  c = lax.axis_index("c")
  s = lax.axis_index("s")
  isem = sems[:IDXD]
  gsem = sems[IDXD:IDXD + DEPTH]
  ssem = sems[IDXD + DEPTH:]

  def fill_fbuf(val):
    def body(r, _):
      fbuf[r] = jnp.full((16,), val, jnp.float32)
      return 0
    lax.fori_loop(0, FC, body, 0)

  def zero_acc():
    fill_fbuf(0.0)
    def body(t, _):
      base = s * RPS + t * FC
      pltpu.sync_copy(fbuf, acc.at[pl.ds(base, FC)])
      return 0
    lax.fori_loop(0, FITER, body, 0)

  def idx_args(g, p):
    return [(src2d.at[pl.ds(s * E16 + g * GEDGE, GEDGE)], sidx.at[p], isem[p]),
            (dst2d.at[pl.ds(s * E16 + g * GEDGE, GEDGE)], didx.at[p], isem[p])]

  def load_idx(g, p):
    for a in idx_args(g, p):
      pltpu.async_copy(*a)

  def drain_idx(g, p):
    for a in idx_args(g, p):
      pltpu.make_async_copy(*a).wait()

  def gat_args(h_hbm, p, ip):
    return (h_hbm.at[sidx.at[ip]], rows.at[p], gsem[p])

  def sca_args(p, ip):
    return (rows.at[p], acc.at[didx.at[ip]], ssem[p])

  def edge_pass(h_hbm, glo, ng):
    if h_hbm is None:
      for p in range(DEPTH):
        def body(r, _, p=p):
          rows[p, r] = jnp.full((16,), 1.0, jnp.float32)
          return 0
        lax.fori_loop(0, GEDGE, body, 0)
    for g0 in range(DEPTH):
      load_idx(glo + g0, g0)
    for g0 in range(GLEAD):
      drain_idx(glo + g0, g0)
      if h_hbm is not None:
        pltpu.async_copy(*gat_args(h_hbm, g0, g0))

    def outer(G, _):
      for u in range(IDXD):
        g = IDXD * G + u
        r = u % DEPTH
        qg = (u + GLEAD) % DEPTH
        ig = (u + GLEAD) % IDXD
        i4 = (u + ILEAD) % IDXD

        @pl.when(jnp.logical_and(g + GLEAD < ng, g + GLEAD >= DEPTH))
        def _():
          pltpu.make_async_copy(*sca_args(qg, 0)).wait()

        @pl.when(g + GLEAD < ng)
        def _():
          drain_idx(glo + g + GLEAD, ig)
          if h_hbm is not None:
            pltpu.async_copy(*gat_args(h_hbm, qg, ig))

        if h_hbm is not None:
          pltpu.make_async_copy(*gat_args(h_hbm, r, u)).wait()
        pltpu.async_copy(*sca_args(r, u), add=True)

        @pl.when(g + ILEAD < ng)
        def _():
          load_idx(glo + g + ILEAD, i4)
      return 0
    lax.fori_loop(0, ng // IDXD, outer, 0)
    for p in range(DEPTH):
      pltpu.make_async_copy(*sca_args(p, 0)).wait()

  def flush(out_hbm):
    def body(t, _):
      base = s * RPS + t * FC
      pltpu.sync_copy(acc.at[pl.ds(base, FC)], fbuf)
      pltpu.sync_copy(fbuf, out_hbm.at[pl.ds(base, FC)])
      return 0
    lax.fori_loop(0, FITER, body, 0)

  for core_id in range(2):
    def run_core(core_id=core_id):
      for (h_idx, out_idx, glo, ng) in schedule[core_id]:
        zero_acc()
        plsc.subcore_barrier()
        edge_pass(None if h_idx is None else hs[h_idx], glo, ng)
        plsc.subcore_barrier()
        flush(outs[out_idx])
        plsc.subcore_barrier()
    pl.when(c == core_id)(run_core)


def _make_sc(schedule, n_out):
  outs = [jax.ShapeDtypeStruct((NPAD, 16), jnp.float32)] * n_out
  return pl.kernel(
      functools.partial(_sc_body, schedule, n_out),
      out_type=outs,
      mesh=plsc.VectorSubcoreMesh(core_axis_name="c", subcore_axis_name="s"),
      compiler_params=pltpu.CompilerParams(use_tc_tiling_on_sc=False),
      scratch_types=[
          pltpu.VMEM_SHARED((NPAD, 16), jnp.float32),
          pltpu.VMEM((IDXD, GEDGE), jnp.int32),
          pltpu.VMEM((IDXD, GEDGE), jnp.int32),
          pltpu.VMEM((DEPTH, GEDGE, 16), jnp.float32),
          pltpu.VMEM((FC, 16), jnp.float32),
      ] + [pltpu.SemaphoreType.DMA] * (IDXD + 2 * DEPTH),
  )


def _k1_body(x_ref, xd_ref, xst_ref, emb_ref, h0_ref, hA_ref, hB_ref, hC_ref):
  xst = xst_ref[...]
  oh = (lax.broadcasted_iota(jnp.int32, (BN, 256), 1) == xst).astype(jnp.float32)
  emb = jnp.dot(oh, emb_ref[...], preferred_element_type=jnp.float32)
  h = jnp.concatenate(
      [x_ref[...], xd_ref[...], emb, jnp.zeros((BN, 2), jnp.float32)], axis=1)
  h0_ref[...] = h
  hA_ref[...] = h[:, 0:16]
  hB_ref[...] = h[:, 16:32]
  hC_ref[...] = h[:, 32:48]


def _combine(aA, aB, aC0, aC1, dg, h, WlT, bl, WrT, g, b):
  agg = jnp.concatenate([aA[...], aB[...], aC0[...] + aC1[...]], axis=1)
  invd = 1.0 / jnp.maximum(dg[...][:, 0:1], 1.0)
  z = (jnp.dot(agg * invd, WlT[...], preferred_element_type=jnp.float32)
       + bl[...]
       + jnp.dot(h[...], WrT[...], preferred_element_type=jnp.float32))
  mu = jnp.mean(z, axis=1, keepdims=True)
  var = jnp.mean((z - mu) ** 2, axis=1, keepdims=True)
  zn = (z - mu) * lax.rsqrt(var + 1e-5) * g[...] + b[...]
  return jnp.maximum(zn, 0.0)


def _k2_body(aA, aB, aC0, aC1, dg, h0, WlT, bl, WrT, g, b,
             h1_ref, oA_ref, oB_ref, oC_ref):
  h1 = _combine(aA, aB, aC0, aC1, dg, h0, WlT, bl, WrT, g, b)
  h1_ref[...] = h1
  oA_ref[...] = h1[:, 0:16]
  oB_ref[...] = h1[:, 16:32]
  oC_ref[...] = h1[:, 32:48]


def _k3_body(aA, aB, aC0, aC1, dg, h1, WlT, bl, WrT, g, b, bt_ref,
             W1T, b1, W2T, b2, W3T, b3, out_ref, ssum, smax, scnt):
  i = pl.program_id(0)

  @pl.when(i == 0)
  def _init():
    ssum[...] = jnp.zeros((NUM_GRAPHS, H), jnp.float32)
    smax[...] = jnp.full((NUM_GRAPHS, H), -jnp.inf, jnp.float32)
    scnt[...] = jnp.zeros((NUM_GRAPHS, 1), jnp.float32)

  h2 = _combine(aA, aB, aC0, aC1, dg, h1, WlT, bl, WrT, g, b)
  bt = bt_ref[...]
  m = bt == lax.broadcasted_iota(jnp.int32, (BN, NUM_GRAPHS), 1)
  mf = m.astype(jnp.float32)
  ssum[...] += lax.dot_general(mf, h2, (((0,), (0,)), ((), ())),
                               preferred_element_type=jnp.float32)
  scnt[...] += jnp.sum(mf, axis=0).reshape(NUM_GRAPHS, 1)
  for gi in range(NUM_GRAPHS):
    v = jnp.max(jnp.where(m[:, gi:gi + 1], h2, -jnp.inf), axis=0)
    smax[gi:gi + 1, :] = jnp.maximum(smax[gi:gi + 1, :], v.reshape(1, H))

  @pl.when(i == GRID - 1)
  def _fin():
    mean = ssum[...] / jnp.maximum(scnt[...], 1.0)
    zz = jnp.concatenate([mean, smax[...]], axis=1)
    zz = jnp.maximum(jnp.dot(zz, W1T[...], preferred_element_type=jnp.float32) + b1[...], 0.0)
    zz = jnp.maximum(jnp.dot(zz, W2T[...], preferred_element_type=jnp.float32) + b2[...], 0.0)
    zz = jnp.maximum(jnp.dot(zz, W3T[...], preferred_element_type=jnp.float32) + b3[...], 0.0)
    out_ref[...] = zz


def _row_spec(w):
  return pl.BlockSpec((BN, w), lambda i: (i, 0))


def _full_spec(shape):
  return pl.BlockSpec(shape, lambda i: tuple(0 for _ in shape))


def kernel(x, edge_index, edge_attr, xdims, xsttype, batch, st_emb,
           Wl1, bl1, Wr1, g1, beta1, Wl2, bl2, Wr2, g2, beta2,
           Wfc1, bfc1, Wfc2, bfc2, Wfc3, bfc3):
  f32 = jnp.float32
  srcp = jnp.concatenate([edge_index[0], jnp.zeros((EPAD - E,), jnp.int32)])
  dstp = jnp.concatenate([edge_index[1], jnp.full((EPAD - E,), N, jnp.int32)])
  src2d = srcp
  dst2d = dstp
  Wl1T = jnp.pad(Wl1, ((0, 0), (0, 2))).T
  Wr1T = jnp.pad(Wr1, ((0, 0), (0, 2))).T
  Wl2T = Wl2.T
  Wr2T = Wr2.T
  row = lambda v: v.reshape(1, -1)

  k1 = pl.pallas_call(
      _k1_body,
      grid=(GRID,),
      in_specs=[_row_spec(32), _row_spec(2), _row_spec(1), _full_spec((256, 12))],
      out_specs=[_row_spec(48), _row_spec(16), _row_spec(16), _row_spec(16)],
      out_shape=[jax.ShapeDtypeStruct((N, 48), f32)] +
                [jax.ShapeDtypeStruct((N, 16), f32)] * 3,
  )
  h0, hA, hB, hC = k1(x, xdims, xsttype.reshape(N, 1), st_emb)

  sched1 = (((0, 0, 0, GROUPS), (2, 2, 0, 296)),
            ((1, 1, 0, GROUPS), (None, 4, 0, GROUPS), (2, 3, 296, 96)))
  sc1 = _make_sc(sched1, 5)
  aggA, aggB, aggC0, aggC1, deg = sc1(hA, hB, hC, src2d, dst2d)

  wspec = [_full_spec((48, 48)), _full_spec((1, 48)), _full_spec((48, 48)),
           _full_spec((1, 48)), _full_spec((1, 48))]
  k2 = pl.pallas_call(
      _k2_body,
      grid=(GRID,),
      in_specs=[_row_spec(16)] * 4 + [_row_spec(16), _row_spec(48)] + wspec,
      out_specs=[_row_spec(48), _row_spec(16), _row_spec(16), _row_spec(16)],
      out_shape=[jax.ShapeDtypeStruct((N, 48), f32)] +
                [jax.ShapeDtypeStruct((N, 16), f32)] * 3,
  )
  h1, h1A, h1B, h1C = k2(aggA, aggB, aggC0, aggC1, deg, h0,
                         Wl1T, row(bl1), Wr1T, row(g1), row(beta1))

  sched2 = (((0, 0, 0, GROUPS), (2, 2, 0, 200)),
            ((1, 1, 0, GROUPS), (2, 3, 200, 192)))
  sc2 = _make_sc(sched2, 4)
  agg2A, agg2B, agg2C0, agg2C1 = sc2(h1A, h1B, h1C, src2d, dst2d)

  k3 = pl.pallas_call(
      _k3_body,
      grid=(GRID,),
      in_specs=([_row_spec(16)] * 4 + [_row_spec(16), _row_spec(48)] + wspec +
                [_row_spec(1),
                 _full_spec((96, 50)), _full_spec((1, 50)),
                 _full_spec((50, 50)), _full_spec((1, 50)),
                 _full_spec((50, 10)), _full_spec((1, 10))]),
      out_specs=pl.BlockSpec((NUM_GRAPHS, 10), lambda i: (0, 0)),
      out_shape=jax.ShapeDtypeStruct((NUM_GRAPHS, 10), f32),
      scratch_shapes=[pltpu.VMEM((NUM_GRAPHS, H), f32),
                      pltpu.VMEM((NUM_GRAPHS, H), f32),
                      pltpu.VMEM((NUM_GRAPHS, 1), f32)],
  )
  out = k3(agg2A, agg2B, agg2C0, agg2C1, deg, h1,
           Wl2T, row(bl2), Wr2T, row(g2), row(beta2), batch.reshape(N, 1),
           Wfc1.T, row(bfc1), Wfc2.T, row(bfc2), Wfc3.T, row(bfc3))
  return out


# speedup vs baseline: 1.5305x; 1.0333x over previous
import functools

import jax
import jax.numpy as jnp
from jax import lax
from jax.experimental import pallas as pl
from jax.experimental.pallas import tpu as pltpu
from jax.experimental.pallas import tpu_sc as plsc

N = 100000
E = 1600000
NUM_GRAPHS = 64
H = 48
BN = 400
GRID = N // BN
CHUNK = 128
NSUB = 16
E16 = 100352
EPAD = E16 * NSUB
NITER = E16 // CHUNK
CROWS = 2
GEDGE = CROWS * CHUNK
GROUPS = NITER // CROWS
DEPTH = 4
IDXD = 2 * DEPTH
GLEAD = 2
ILEAD = 6
RPS = 6400
NPAD = RPS * NSUB
FC = 256
FITER = RPS // FC


def _sc_body(schedule, n_out, *args):
  (hA, hB, hC, src2d, dst2d) = args[:5]
  outs = args[5:5 + n_out]
  (acc, sidx, didx, rows, fbuf, *sems) = args[5 + n_out:]
  hs = [hA, hB, hC]
  c = lax.axis_index("c")
  s = lax.axis_index("s")
  isem = sems[:IDXD]
  gsem = sems[IDXD:IDXD + DEPTH]
  ssem = sems[IDXD + DEPTH:]

  def fill_fbuf(val):
    def body(r, _):
      fbuf[r] = jnp.full((16,), val, jnp.float32)
      return 0
    lax.fori_loop(0, FC, body, 0)

  def zero_acc():
    fill_fbuf(0.0)
    def body(t, _):
      base = s * RPS + t * FC
      pltpu.sync_copy(fbuf, acc.at[pl.ds(base, FC)])
      return 0
    lax.fori_loop(0, FITER, body, 0)

  def idx_args(g, p):
    return [(src2d.at[pl.ds(s * E16 + g * GEDGE, GEDGE)], sidx.at[p], isem[p]),
            (dst2d.at[pl.ds(s * E16 + g * GEDGE, GEDGE)], didx.at[p], isem[p])]

  def load_idx(g, p):
    for a in idx_args(g, p):
      pltpu.async_copy(*a)

  def drain_idx(g, p):
    for a in idx_args(g, p):
      pltpu.make_async_copy(*a).wait()

  def gat_args(h_hbm, p, ip):
    return (h_hbm.at[sidx.at[ip]], rows.at[p], gsem[p])

  def sca_args(p, ip):
    return (rows.at[p], acc.at[didx.at[ip]], ssem[p])

  def edge_pass(h_hbm, glo, ng):
    if h_hbm is None:
      for p in range(DEPTH):
        def body(r, _, p=p):
          rows[p, r] = jnp.full((16,), 1.0, jnp.float32)
          return 0
        lax.fori_loop(0, GEDGE, body, 0)
    for g0 in range(ILEAD):
      load_idx(glo + g0, g0)
    for g0 in range(GLEAD):
      drain_idx(glo + g0, g0)
      if h_hbm is not None:
        pltpu.async_copy(*gat_args(h_hbm, g0, g0))

    def outer(G, _):
      for u in range(IDXD):
        g = IDXD * G + u
        r = u % DEPTH
        qg = (u + GLEAD) % DEPTH
        ig = (u + GLEAD) % IDXD
        i4 = (u + ILEAD) % IDXD

        @pl.when(jnp.logical_and(g + GLEAD < ng, g + GLEAD >= DEPTH))
        def _():
          pltpu.make_async_copy(*sca_args(qg, 0)).wait()

        @pl.when(g + GLEAD < ng)
        def _():
          drain_idx(glo + g + GLEAD, ig)
          if h_hbm is not None:
            pltpu.async_copy(*gat_args(h_hbm, qg, ig))

        if h_hbm is not None:
          pltpu.make_async_copy(*gat_args(h_hbm, r, u)).wait()
        pltpu.async_copy(*sca_args(r, u), add=True)

        @pl.when(g + ILEAD < ng)
        def _():
          load_idx(glo + g + ILEAD, i4)
      return 0
    lax.fori_loop(0, ng // IDXD, outer, 0)
    for p in range(DEPTH):
      pltpu.make_async_copy(*sca_args(p, 0)).wait()

  def flush(out_hbm):
    def body(t, _):
      base = s * RPS + t * FC
      pltpu.sync_copy(acc.at[pl.ds(base, FC)], fbuf)
      pltpu.sync_copy(fbuf, out_hbm.at[pl.ds(base, FC)])
      return 0
    lax.fori_loop(0, FITER, body, 0)

  for core_id in range(2):
    def run_core(core_id=core_id):
      for (h_idx, out_idx, glo, ng) in schedule[core_id]:
        zero_acc()
        plsc.subcore_barrier()
        edge_pass(None if h_idx is None else hs[h_idx], glo, ng)
        plsc.subcore_barrier()
        flush(outs[out_idx])
        plsc.subcore_barrier()
    pl.when(c == core_id)(run_core)


def _make_sc(schedule, n_out):
  outs = [jax.ShapeDtypeStruct((NPAD, 16), jnp.float32)] * n_out
  return pl.kernel(
      functools.partial(_sc_body, schedule, n_out),
      out_type=outs,
      mesh=plsc.VectorSubcoreMesh(core_axis_name="c", subcore_axis_name="s"),
      compiler_params=pltpu.CompilerParams(use_tc_tiling_on_sc=False),
      scratch_types=[
          pltpu.VMEM_SHARED((NPAD, 16), jnp.float32),
          pltpu.VMEM((IDXD, GEDGE), jnp.int32),
          pltpu.VMEM((IDXD, GEDGE), jnp.int32),
          pltpu.VMEM((DEPTH, GEDGE, 16), jnp.float32),
          pltpu.VMEM((FC, 16), jnp.float32),
      ] + [pltpu.SemaphoreType.DMA] * (IDXD + 2 * DEPTH),
  )


def _k1_body(x_ref, xd_ref, xst_ref, emb_ref, h0_ref, hA_ref, hB_ref, hC_ref):
  xst = xst_ref[...]
  oh = (lax.broadcasted_iota(jnp.int32, (BN, 256), 1) == xst).astype(jnp.float32)
  emb = jnp.dot(oh, emb_ref[...], preferred_element_type=jnp.float32)
  h = jnp.concatenate(
      [x_ref[...], xd_ref[...], emb, jnp.zeros((BN, 2), jnp.float32)], axis=1)
  h0_ref[...] = h
  hA_ref[...] = h[:, 0:16]
  hB_ref[...] = h[:, 16:32]
  hC_ref[...] = h[:, 32:48]


def _combine(aA, aB, aC0, aC1, dg, h, WlT, bl, WrT, g, b):
  agg = jnp.concatenate([aA[...], aB[...], aC0[...] + aC1[...]], axis=1)
  invd = 1.0 / jnp.maximum(dg[...][:, 0:1], 1.0)
  z = (jnp.dot(agg * invd, WlT[...], preferred_element_type=jnp.float32)
       + bl[...]
       + jnp.dot(h[...], WrT[...], preferred_element_type=jnp.float32))
  mu = jnp.mean(z, axis=1, keepdims=True)
  var = jnp.mean((z - mu) ** 2, axis=1, keepdims=True)
  zn = (z - mu) * lax.rsqrt(var + 1e-5) * g[...] + b[...]
  return jnp.maximum(zn, 0.0)


def _k2_body(aA, aB, aC0, aC1, dg, h0, WlT, bl, WrT, g, b,
             h1_ref, oA_ref, oB_ref, oC_ref):
  h1 = _combine(aA, aB, aC0, aC1, dg, h0, WlT, bl, WrT, g, b)
  h1_ref[...] = h1
  oA_ref[...] = h1[:, 0:16]
  oB_ref[...] = h1[:, 16:32]
  oC_ref[...] = h1[:, 32:48]


def _k3_body(aA, aB, aC0, aC1, dg, h1, WlT, bl, WrT, g, b, bt_ref,
             W1T, b1, W2T, b2, W3T, b3, out_ref, ssum, smax, scnt):
  i = pl.program_id(0)

  @pl.when(i == 0)
  def _init():
    ssum[...] = jnp.zeros((NUM_GRAPHS, H), jnp.float32)
    smax[...] = jnp.full((NUM_GRAPHS, H), -jnp.inf, jnp.float32)
    scnt[...] = jnp.zeros((NUM_GRAPHS, 1), jnp.float32)

  h2 = _combine(aA, aB, aC0, aC1, dg, h1, WlT, bl, WrT, g, b)
  bt = bt_ref[...]
  m = bt == lax.broadcasted_iota(jnp.int32, (BN, NUM_GRAPHS), 1)
  mf = m.astype(jnp.float32)
  ssum[...] += lax.dot_general(mf, h2, (((0,), (0,)), ((), ())),
                               preferred_element_type=jnp.float32)
  scnt[...] += jnp.sum(mf, axis=0).reshape(NUM_GRAPHS, 1)
  for gi in range(NUM_GRAPHS):
    v = jnp.max(jnp.where(m[:, gi:gi + 1], h2, -jnp.inf), axis=0)
    smax[gi:gi + 1, :] = jnp.maximum(smax[gi:gi + 1, :], v.reshape(1, H))

  @pl.when(i == GRID - 1)
  def _fin():
    mean = ssum[...] / jnp.maximum(scnt[...], 1.0)
    zz = jnp.concatenate([mean, smax[...]], axis=1)
    zz = jnp.maximum(jnp.dot(zz, W1T[...], preferred_element_type=jnp.float32) + b1[...], 0.0)
    zz = jnp.maximum(jnp.dot(zz, W2T[...], preferred_element_type=jnp.float32) + b2[...], 0.0)
    zz = jnp.maximum(jnp.dot(zz, W3T[...], preferred_element_type=jnp.float32) + b3[...], 0.0)
    out_ref[...] = zz


def _row_spec(w):
  return pl.BlockSpec((BN, w), lambda i: (i, 0))


def _full_spec(shape):
  return pl.BlockSpec(shape, lambda i: tuple(0 for _ in shape))


def kernel(x, edge_index, edge_attr, xdims, xsttype, batch, st_emb,
           Wl1, bl1, Wr1, g1, beta1, Wl2, bl2, Wr2, g2, beta2,
           Wfc1, bfc1, Wfc2, bfc2, Wfc3, bfc3):
  f32 = jnp.float32
  srcp = jnp.concatenate([edge_index[0], jnp.zeros((EPAD - E,), jnp.int32)])
  trash = N + (jnp.arange(EPAD - E, dtype=jnp.int32) % (NPAD - N))
  dstp = jnp.concatenate([edge_index[1], trash])
  src2d = srcp
  dst2d = dstp
  Wl1T = jnp.pad(Wl1, ((0, 0), (0, 2))).T
  Wr1T = jnp.pad(Wr1, ((0, 0), (0, 2))).T
  Wl2T = Wl2.T
  Wr2T = Wr2.T
  row = lambda v: v.reshape(1, -1)

  k1 = pl.pallas_call(
      _k1_body,
      grid=(GRID,),
      in_specs=[_row_spec(32), _row_spec(2), _row_spec(1), _full_spec((256, 12))],
      out_specs=[_row_spec(48), _row_spec(16), _row_spec(16), _row_spec(16)],
      out_shape=[jax.ShapeDtypeStruct((N, 48), f32)] +
                [jax.ShapeDtypeStruct((N, 16), f32)] * 3,
  )
  h0, hA, hB, hC = k1(x, xdims, xsttype.reshape(N, 1), st_emb)

  sched1 = (((0, 0, 0, GROUPS), (2, 2, 0, 296)),
            ((1, 1, 0, GROUPS), (None, 4, 0, GROUPS), (2, 3, 296, 96)))
  sc1 = _make_sc(sched1, 5)
  aggA, aggB, aggC0, aggC1, deg = sc1(hA, hB, hC, src2d, dst2d)

  wspec = [_full_spec((48, 48)), _full_spec((1, 48)), _full_spec((48, 48)),
           _full_spec((1, 48)), _full_spec((1, 48))]
  k2 = pl.pallas_call(
      _k2_body,
      grid=(GRID,),
      in_specs=[_row_spec(16)] * 4 + [_row_spec(16), _row_spec(48)] + wspec,
      out_specs=[_row_spec(48), _row_spec(16), _row_spec(16), _row_spec(16)],
      out_shape=[jax.ShapeDtypeStruct((N, 48), f32)] +
                [jax.ShapeDtypeStruct((N, 16), f32)] * 3,
  )
  h1, h1A, h1B, h1C = k2(aggA, aggB, aggC0, aggC1, deg, h0,
                         Wl1T, row(bl1), Wr1T, row(g1), row(beta1))

  sched2 = (((0, 0, 0, GROUPS), (2, 2, 0, 200)),
            ((1, 1, 0, GROUPS), (2, 3, 200, 192)))
  sc2 = _make_sc(sched2, 4)
  agg2A, agg2B, agg2C0, agg2C1 = sc2(h1A, h1B, h1C, src2d, dst2d)

  k3 = pl.pallas_call(
      _k3_body,
      grid=(GRID,),
      in_specs=([_row_spec(16)] * 4 + [_row_spec(16), _row_spec(48)] + wspec +
                [_row_spec(1),
                 _full_spec((96, 50)), _full_spec((1, 50)),
                 _full_spec((50, 50)), _full_spec((1, 50)),
                 _full_spec((50, 10)), _full_spec((1, 10))]),
      out_specs=pl.BlockSpec((NUM_GRAPHS, 10), lambda i: (0, 0)),
      out_shape=jax.ShapeDtypeStruct((NUM_GRAPHS, 10), f32),
      scratch_shapes=[pltpu.VMEM((NUM_GRAPHS, H), f32),
                      pltpu.VMEM((NUM_GRAPHS, H), f32),
                      pltpu.VMEM((NUM_GRAPHS, 1), f32)],
  )
  out = k3(agg2A, agg2B, agg2C0, agg2C1, deg, h1,
           Wl2T, row(bl2), Wr2T, row(g2), row(beta2), batch.reshape(N, 1),
           Wfc1.T, row(bfc1), Wfc2.T, row(bfc2), Wfc3.T, row(bfc3))
  return out


# speedup vs baseline: 2.0437x; 1.3353x over previous
import functools

import jax
import jax.numpy as jnp
from jax import lax
from jax.experimental import pallas as pl
from jax.experimental.pallas import tpu as pltpu
from jax.experimental.pallas import tpu_sc as plsc

N = 100000
E = 1600000
NUM_GRAPHS = 64
H = 48
BN = 400
GRID = N // BN
CHUNK = 128
NSUB = 16
E16 = 100352
EPAD = E16 * NSUB
NITER = E16 // CHUNK
CROWS = 2
GEDGE = CROWS * CHUNK
GROUPS = NITER // CROWS
DEPTH = 4
IDXD = 2 * DEPTH
GLEAD = 2
ILEAD = 6
RPS = 6400
NPAD = RPS * NSUB
FC = 256
FITER = RPS // FC


def _sc_body(schedule, n_out, *args):
  (hA, hB, hC, src2d, dst2d) = args[:5]
  outs = args[5:5 + n_out]
  (acc, sidx, didx, rows, fbuf, *sems) = args[5 + n_out:]
  hs = [hA, hB, hC]
  c = lax.axis_index("c")
  s = lax.axis_index("s")
  isem = sems[:IDXD]
  gsem = sems[IDXD:IDXD + DEPTH]
  ssem = sems[IDXD + DEPTH:]

  def fill_fbuf(val):
    def body(r, _):
      fbuf[r] = jnp.full((16,), val, jnp.float32)
      return 0
    lax.fori_loop(0, FC, body, 0)

  def zero_acc():
    fill_fbuf(0.0)
    def body(t, _):
      base = s * RPS + t * FC
      pltpu.sync_copy(fbuf, acc.at[pl.ds(base, FC)])
      return 0
    lax.fori_loop(0, FITER, body, 0)

  def idx_args(g, p):
    return [(src2d.at[pl.ds(s * E16 + g * GEDGE, GEDGE)], sidx.at[p], isem[p]),
            (dst2d.at[pl.ds(s * E16 + g * GEDGE, GEDGE)], didx.at[p], isem[p])]

  def load_idx(g, p):
    for a in idx_args(g, p):
      pltpu.async_copy(*a)

  def drain_idx(g, p):
    for a in idx_args(g, p):
      pltpu.make_async_copy(*a).wait()

  def gat_args(h_hbm, p, ip):
    return (h_hbm.at[sidx.at[ip]], rows.at[p], gsem[p])

  def sca_args(p, ip):
    return (rows.at[p], acc.at[didx.at[ip]], ssem[p])

  def edge_pass(h_hbm, glo, ng):
    if h_hbm is None:
      for p in range(DEPTH):
        def body(r, _, p=p):
          rows[p, r] = jnp.full((16,), 1.0, jnp.float32)
          return 0
        lax.fori_loop(0, GEDGE, body, 0)
    for g0 in range(ILEAD):
      load_idx(glo + g0, g0)
    for g0 in range(GLEAD):
      drain_idx(glo + g0, g0)
      if h_hbm is not None:
        pltpu.async_copy(*gat_args(h_hbm, g0, g0))

    def outer(G, _):
      for u in range(IDXD):
        g = IDXD * G + u
        r = u % DEPTH
        qg = (u + GLEAD) % DEPTH
        ig = (u + GLEAD) % IDXD
        i4 = (u + ILEAD) % IDXD

        @pl.when(jnp.logical_and(g + GLEAD < ng, g + GLEAD >= DEPTH))
        def _():
          pltpu.make_async_copy(*sca_args(qg, 0)).wait()

        @pl.when(g + GLEAD < ng)
        def _():
          drain_idx(glo + g + GLEAD, ig)
          if h_hbm is not None:
            pltpu.async_copy(*gat_args(h_hbm, qg, ig))

        if h_hbm is not None:
          pltpu.make_async_copy(*gat_args(h_hbm, r, u)).wait()
        pltpu.async_copy(*sca_args(r, u), add=True)

        @pl.when(g + ILEAD < ng)
        def _():
          load_idx(glo + g + ILEAD, i4)
      return 0
    lax.fori_loop(0, ng // IDXD, outer, 0)
    for p in range(DEPTH):
      pltpu.make_async_copy(*sca_args(p, 0)).wait()

  def flush(out_hbm):
    def body(t, _):
      base = s * RPS + t * FC
      pltpu.sync_copy(acc.at[pl.ds(base, FC)], fbuf)
      pltpu.sync_copy(fbuf, out_hbm.at[pl.ds(base, FC)])
      return 0
    lax.fori_loop(0, FITER, body, 0)

  for core_id in range(2):
    def run_core(core_id=core_id):
      for (h_idx, out_idx, glo, ng) in schedule[core_id]:
        zero_acc()
        plsc.subcore_barrier()
        edge_pass(None if h_idx is None else hs[h_idx], glo, ng)
        plsc.subcore_barrier()
        flush(outs[out_idx])
        plsc.subcore_barrier()
    pl.when(c == core_id)(run_core)


def _make_sc(schedule, n_out):
  outs = [jax.ShapeDtypeStruct((NPAD, 16), jnp.float32)] * n_out
  return pl.kernel(
      functools.partial(_sc_body, schedule, n_out),
      out_type=outs,
      mesh=plsc.VectorSubcoreMesh(core_axis_name="c", subcore_axis_name="s"),
      compiler_params=pltpu.CompilerParams(use_tc_tiling_on_sc=False),
      scratch_types=[
          pltpu.VMEM_SHARED((NPAD, 16), jnp.float32),
          pltpu.VMEM((IDXD, GEDGE), jnp.int32),
          pltpu.VMEM((IDXD, GEDGE), jnp.int32),
          pltpu.VMEM((DEPTH, GEDGE, 16), jnp.float32),
          pltpu.VMEM((FC, 16), jnp.float32),
      ] + [pltpu.SemaphoreType.DMA] * (IDXD + 2 * DEPTH),
  )


def _k1_body(x_ref, xd_ref, xst_ref, emb_ref, h0_ref, hA_ref, hB_ref, hC_ref):
  xst = xst_ref[...]
  oh = (lax.broadcasted_iota(jnp.int32, (BN, 256), 1) == xst).astype(jnp.float32)
  emb = jnp.dot(oh, emb_ref[...], preferred_element_type=jnp.float32)
  h = jnp.concatenate(
      [x_ref[...], xd_ref[...], emb, jnp.zeros((BN, 2), jnp.float32)], axis=1)
  h0_ref[...] = h
  hA_ref[...] = h[:, 0:16]
  hB_ref[...] = h[:, 16:32]
  hC_ref[...] = h[:, 32:48]


def _combine(aA, aB, aC0, aC1, dg, h, WlT, bl, WrT, g, b):
  agg = jnp.concatenate([aA[...], aB[...], aC0[...] + aC1[...]], axis=1)
  invd = 1.0 / jnp.maximum(dg[...][:, 0:1], 1.0)
  z = (jnp.dot(agg * invd, WlT[...], preferred_element_type=jnp.float32)
       + bl[...]
       + jnp.dot(h[...], WrT[...], preferred_element_type=jnp.float32))
  mu = jnp.mean(z, axis=1, keepdims=True)
  var = jnp.mean((z - mu) ** 2, axis=1, keepdims=True)
  zn = (z - mu) * lax.rsqrt(var + 1e-5) * g[...] + b[...]
  return jnp.maximum(zn, 0.0)


def _k2_body(aA, aB, aC0, aC1, dg, h0, WlT, bl, WrT, g, b,
             h1_ref, oA_ref, oB_ref, oC_ref):
  h1 = _combine(aA, aB, aC0, aC1, dg, h0, WlT, bl, WrT, g, b)
  h1_ref[...] = h1
  oA_ref[...] = h1[:, 0:16]
  oB_ref[...] = h1[:, 16:32]
  oC_ref[...] = h1[:, 32:48]


def _k3_body(aA, aB, aC0, aC1, dg, h1, WlT, bl, WrT, g, b, bt_ref,
             W1T, b1, W2T, b2, W3T, b3, out_ref, ssum, smax, scnt):
  i = pl.program_id(0)

  @pl.when(i == 0)
  def _init():
    ssum[...] = jnp.zeros((NUM_GRAPHS, H), jnp.float32)
    smax[...] = jnp.full((NUM_GRAPHS, H), -jnp.inf, jnp.float32)
    scnt[...] = jnp.zeros((NUM_GRAPHS, 1), jnp.float32)

  h2 = _combine(aA, aB, aC0, aC1, dg, h1, WlT, bl, WrT, g, b)
  bt = bt_ref[...]
  m = bt == lax.broadcasted_iota(jnp.int32, (BN, NUM_GRAPHS), 1)
  mf = m.astype(jnp.float32)
  ssum[...] += lax.dot_general(mf, h2, (((0,), (0,)), ((), ())),
                               preferred_element_type=jnp.float32)
  scnt[...] += jnp.sum(mf, axis=0).reshape(NUM_GRAPHS, 1)
  g_lo = bt[0, 0]
  g_hi = bt[BN - 1, 0]

  @pl.when(g_lo == g_hi)
  def _single():
    bm = jnp.max(h2, axis=0).reshape(1, H)
    smax[pl.ds(g_lo, 1), :] = jnp.maximum(smax[pl.ds(g_lo, 1), :], bm)

  @pl.when(g_lo != g_hi)
  def _multi():
    for gi in range(NUM_GRAPHS):
      @pl.when(jnp.logical_and(gi >= g_lo, gi <= g_hi))
      def _(gi=gi):
        v = jnp.max(jnp.where(m[:, gi:gi + 1], h2, -jnp.inf), axis=0)
        smax[gi:gi + 1, :] = jnp.maximum(smax[gi:gi + 1, :], v.reshape(1, H))

  @pl.when(i == GRID - 1)
  def _fin():
    mean = ssum[...] / jnp.maximum(scnt[...], 1.0)
    zz = jnp.concatenate([mean, smax[...]], axis=1)
    zz = jnp.maximum(jnp.dot(zz, W1T[...], preferred_element_type=jnp.float32) + b1[...], 0.0)
    zz = jnp.maximum(jnp.dot(zz, W2T[...], preferred_element_type=jnp.float32) + b2[...], 0.0)
    zz = jnp.maximum(jnp.dot(zz, W3T[...], preferred_element_type=jnp.float32) + b3[...], 0.0)
    out_ref[...] = zz


def _row_spec(w):
  return pl.BlockSpec((BN, w), lambda i: (i, 0))


def _full_spec(shape):
  return pl.BlockSpec(shape, lambda i: tuple(0 for _ in shape))


def kernel(x, edge_index, edge_attr, xdims, xsttype, batch, st_emb,
           Wl1, bl1, Wr1, g1, beta1, Wl2, bl2, Wr2, g2, beta2,
           Wfc1, bfc1, Wfc2, bfc2, Wfc3, bfc3):
  f32 = jnp.float32
  srcp = jnp.concatenate([edge_index[0], jnp.zeros((EPAD - E,), jnp.int32)])
  trash = N + (jnp.arange(EPAD - E, dtype=jnp.int32) % (NPAD - N))
  dstp = jnp.concatenate([edge_index[1], trash])
  src2d = srcp
  dst2d = dstp
  Wl1T = jnp.pad(Wl1, ((0, 0), (0, 2))).T
  Wr1T = jnp.pad(Wr1, ((0, 0), (0, 2))).T
  Wl2T = Wl2.T
  Wr2T = Wr2.T
  row = lambda v: v.reshape(1, -1)

  k1 = pl.pallas_call(
      _k1_body,
      grid=(GRID,),
      in_specs=[_row_spec(32), _row_spec(2), _row_spec(1), _full_spec((256, 12))],
      out_specs=[_row_spec(48), _row_spec(16), _row_spec(16), _row_spec(16)],
      out_shape=[jax.ShapeDtypeStruct((N, 48), f32)] +
                [jax.ShapeDtypeStruct((N, 16), f32)] * 3,
  )
  h0, hA, hB, hC = k1(x, xdims, xsttype.reshape(N, 1), st_emb)

  sched1 = (((0, 0, 0, GROUPS), (2, 2, 0, 296)),
            ((1, 1, 0, GROUPS), (None, 4, 0, GROUPS), (2, 3, 296, 96)))
  sc1 = _make_sc(sched1, 5)
  aggA, aggB, aggC0, aggC1, deg = sc1(hA, hB, hC, src2d, dst2d)

  wspec = [_full_spec((48, 48)), _full_spec((1, 48)), _full_spec((48, 48)),
           _full_spec((1, 48)), _full_spec((1, 48))]
  k2 = pl.pallas_call(
      _k2_body,
      grid=(GRID,),
      in_specs=[_row_spec(16)] * 4 + [_row_spec(16), _row_spec(48)] + wspec,
      out_specs=[_row_spec(48), _row_spec(16), _row_spec(16), _row_spec(16)],
      out_shape=[jax.ShapeDtypeStruct((N, 48), f32)] +
                [jax.ShapeDtypeStruct((N, 16), f32)] * 3,
  )
  h1, h1A, h1B, h1C = k2(aggA, aggB, aggC0, aggC1, deg, h0,
                         Wl1T, row(bl1), Wr1T, row(g1), row(beta1))

  sched2 = (((0, 0, 0, GROUPS), (2, 2, 0, 200)),
            ((1, 1, 0, GROUPS), (2, 3, 200, 192)))
  sc2 = _make_sc(sched2, 4)
  agg2A, agg2B, agg2C0, agg2C1 = sc2(h1A, h1B, h1C, src2d, dst2d)

  k3 = pl.pallas_call(
      _k3_body,
      grid=(GRID,),
      in_specs=([_row_spec(16)] * 4 + [_row_spec(16), _row_spec(48)] + wspec +
                [_row_spec(1),
                 _full_spec((96, 50)), _full_spec((1, 50)),
                 _full_spec((50, 50)), _full_spec((1, 50)),
                 _full_spec((50, 10)), _full_spec((1, 10))]),
      out_specs=pl.BlockSpec((NUM_GRAPHS, 10), lambda i: (0, 0)),
      out_shape=jax.ShapeDtypeStruct((NUM_GRAPHS, 10), f32),
      scratch_shapes=[pltpu.VMEM((NUM_GRAPHS, H), f32),
                      pltpu.VMEM((NUM_GRAPHS, H), f32),
                      pltpu.VMEM((NUM_GRAPHS, 1), f32)],
  )
  out = k3(agg2A, agg2B, agg2C0, agg2C1, deg, h1,
           Wl2T, row(bl2), Wr2T, row(g2), row(beta2), batch.reshape(N, 1),
           Wfc1.T, row(bfc1), Wfc2.T, row(bfc2), Wfc3.T, row(bfc3))
  return out
